# baseline (speedup 1.0000x reference)
import functools

import jax
import jax.numpy as jnp
from jax import lax
from jax.experimental import pallas as pl
from jax.experimental.pallas import tpu as pltpu
from jax.experimental.pallas import tpu_sc as plsc

VOCAB = 49408
DIM = 1024
BATCH = 256
SEQ = 77
N_TOK = BATCH * SEQ
NC, NS, LANES = 2, 16, 16
NW = NC * NS
N_PAD = 20480
TOK_PER_W = N_PAD // NW
CHUNK = 64
N_CHUNK = TOK_PER_W // CHUNK
GROUPS = CHUNK // LANES
SLICES = DIM // LANES

_mesh = plsc.VectorSubcoreMesh(core_axis_name="c", subcore_axis_name="s")


def _worker_id():
    return lax.axis_index("s") * NC + lax.axis_index("c")


@functools.partial(
    pl.kernel,
    out_type=(
        jax.ShapeDtypeStruct((NW, LANES), jnp.float32),
        jax.ShapeDtypeStruct((NW, LANES), jnp.float32),
    ),
    mesh=_mesh,
    scratch_types=[
        pltpu.VMEM((CHUNK,), jnp.int32),
        pltpu.VMEM((CHUNK,), jnp.float32),
        pltpu.VMEM((CHUNK,), jnp.float32),
        pltpu.VMEM((CHUNK, DIM), jnp.float32),
        pltpu.VMEM((2, LANES), jnp.float32),
        pltpu.SemaphoreType.DMA,
    ],
)
def _sums_kernel(ids_hbm, mult_hbm, valid_hbm, table_hbm, pz_hbm, pzm_hbm,
                 idx_v, m_v, wz_v, rows_v, stage_v, sem):
    wid = _worker_id()
    base = wid * TOK_PER_W

    def chunk_body(j, carry):
        acc_z, acc_zm = carry
        cbase = base + j * CHUNK
        pltpu.sync_copy(ids_hbm.at[pl.ds(cbase, CHUNK)], idx_v)
        pltpu.sync_copy(mult_hbm.at[pl.ds(cbase, CHUNK)], m_v)
        pltpu.sync_copy(valid_hbm.at[pl.ds(cbase, CHUNK)], wz_v)
        pltpu.async_copy(table_hbm.at[idx_v], rows_v, sem).wait()

        def group_body(g, inner):
            az, azm = inner
            m16 = m_v[pl.ds(g * LANES, LANES)]
            wz16 = wz_v[pl.ds(g * LANES, LANES)]
            for r16 in range(LANES):
                r = g * LANES + r16
                racc = rows_v[r, pl.ds(0, LANES)]
                for k in range(1, SLICES):
                    racc = racc + rows_v[r, pl.ds(k * LANES, LANES)]
                az = az + racc * jnp.full((LANES,), wz16[r16], jnp.float32)
                azm = azm + racc * jnp.full((LANES,), m16[r16], jnp.float32)
            return az, azm

        return lax.fori_loop(0, GROUPS, group_body, (acc_z, acc_zm))

    zero = jnp.zeros((LANES,), jnp.float32)
    acc_z, acc_zm = lax.fori_loop(0, N_CHUNK, chunk_body, (zero, zero))
    stage_v[0, :] = acc_z
    stage_v[1, :] = acc_zm
    pltpu.sync_copy(stage_v.at[0], pz_hbm.at[wid])
    pltpu.sync_copy(stage_v.at[1], pzm_hbm.at[wid])


@functools.partial(
    pl.kernel,
    out_type=jax.ShapeDtypeStruct((N_PAD, DIM), jnp.float32),
    mesh=_mesh,
    scratch_types=[
        pltpu.VMEM((CHUNK,), jnp.int32),
        pltpu.VMEM((CHUNK,), jnp.float32),
        pltpu.VMEM((CHUNK, DIM), jnp.float32),
        pltpu.VMEM((NW, LANES), jnp.float32),
        pltpu.VMEM((NW, LANES), jnp.float32),
        pltpu.SemaphoreType.DMA,
    ],
)
def _scale_kernel(ids_hbm, mult_hbm, table_hbm, pz_hbm, pzm_hbm, out_hbm,
                  idx_v, m_v, rows_v, pz_v, pzm_v, sem):
    wid = _worker_id()
    base = wid * TOK_PER_W

    pltpu.sync_copy(pz_hbm, pz_v)
    pltpu.sync_copy(pzm_hbm, pzm_v)
    vz = pz_v[0, :]
    vzm = pzm_v[0, :]
    for w in range(1, NW):
        vz = vz + pz_v[w, :]
        vzm = vzm + pzm_v[w, :]
    sz = vz[0]
    szm = vzm[0]
    for i in range(1, LANES):
        sz = sz + vz[i]
        szm = szm + vzm[i]
    rsplat = jnp.full((LANES,), sz, jnp.float32) / jnp.full(
        (LANES,), szm, jnp.float32
    )

    def chunk_body(j, carry):
        cbase = base + j * CHUNK
        pltpu.sync_copy(ids_hbm.at[pl.ds(cbase, CHUNK)], idx_v)
        pltpu.sync_copy(mult_hbm.at[pl.ds(cbase, CHUNK)], m_v)
        pltpu.async_copy(table_hbm.at[idx_v], rows_v, sem).wait()

        def group_body(g, inner):
            m16 = m_v[pl.ds(g * LANES, LANES)] * rsplat
            for r16 in range(LANES):
                r = g * LANES + r16
                scale = jnp.full((LANES,), m16[r16], jnp.float32)
                for k in range(SLICES):
                    sl = pl.ds(k * LANES, LANES)
                    rows_v[r, sl] = rows_v[r, sl] * scale
            return inner

        lax.fori_loop(0, GROUPS, group_body, 0)
        pltpu.sync_copy(rows_v, out_hbm.at[pl.ds(cbase, CHUNK)])
        return carry

    lax.fori_loop(0, N_CHUNK, chunk_body, 0)


def kernel(input_ids, weight, batch_multipliers):
    ids = input_ids.reshape(-1).astype(jnp.int32)
    mult = batch_multipliers.reshape(-1).astype(jnp.float32)
    n_extra = N_PAD - N_TOK
    ids_p = jnp.concatenate([ids, jnp.zeros((n_extra,), jnp.int32)])
    mult_p = jnp.concatenate([mult, jnp.zeros((n_extra,), jnp.float32)])
    valid = jnp.concatenate(
        [jnp.ones((N_TOK,), jnp.float32), jnp.zeros((n_extra,), jnp.float32)]
    )
    pz, pzm = _sums_kernel(ids_p, mult_p, valid, weight)
    out = _scale_kernel(ids_p, mult_p, weight, pz, pzm)
    return out[:N_TOK].reshape(BATCH, SEQ, DIM)


# speedup vs baseline: 1.0731x; 1.0731x over previous
import functools

import jax
import jax.numpy as jnp
from jax import lax
from jax.experimental import pallas as pl
from jax.experimental.pallas import tpu as pltpu
from jax.experimental.pallas import tpu_sc as plsc

VOCAB = 49408
DIM = 1024
BATCH = 256
SEQ = 77
N_TOK = BATCH * SEQ
NC, NS, LANES = 2, 16, 16
NW = NC * NS
N_PAD = 20480
TOK_PER_W = N_PAD // NW
CHUNK = 32
NSTEP = TOK_PER_W // CHUNK
GROUPS = CHUNK // LANES
SLICES = DIM // LANES

_mesh = plsc.VectorSubcoreMesh(core_axis_name="c", subcore_axis_name="s")


def _worker_id():
    return lax.axis_index("s") * NC + lax.axis_index("c")


def _gather_chunk(table_hbm, idx_all, j, buf, sem):
    idx_slice = idx_all.at[pl.ds(j * CHUNK, CHUNK)]
    return pltpu.async_copy(table_hbm.at[idx_slice], buf, sem)


@functools.partial(
    pl.kernel,
    out_type=(
        jax.ShapeDtypeStruct((NW, LANES), jnp.float32),
        jax.ShapeDtypeStruct((NW, LANES), jnp.float32),
    ),
    mesh=_mesh,
    scratch_types=[
        pltpu.VMEM((TOK_PER_W,), jnp.int32),
        pltpu.VMEM((TOK_PER_W,), jnp.float32),
        pltpu.VMEM((TOK_PER_W,), jnp.float32),
        pltpu.VMEM((CHUNK, DIM), jnp.float32),
        pltpu.VMEM((CHUNK, DIM), jnp.float32),
        pltpu.VMEM((2, LANES), jnp.float32),
        pltpu.SemaphoreType.DMA,
        pltpu.SemaphoreType.DMA,
    ],
)
def _sums_kernel(ids_hbm, mult_hbm, valid_hbm, table_hbm, pz_hbm, pzm_hbm,
                 idx_all, m_all, wz_all, buf0, buf1, stage_v, sem0, sem1):
    wid = _worker_id()
    base = wid * TOK_PER_W
    pltpu.sync_copy(ids_hbm.at[pl.ds(base, TOK_PER_W)], idx_all)
    pltpu.sync_copy(mult_hbm.at[pl.ds(base, TOK_PER_W)], m_all)
    pltpu.sync_copy(valid_hbm.at[pl.ds(base, TOK_PER_W)], wz_all)

    bufs = (buf0, buf1)
    sems = (sem0, sem1)
    _gather_chunk(table_hbm, idx_all, 0, buf0, sem0)

    def step(j, b, carry):
        acc_z, acc_zm = carry
        buf = bufs[b]
        pltpu.make_async_copy(
            table_hbm.at[idx_all.at[pl.ds(0, CHUNK)]], buf, sems[b]
        ).wait()

        @pl.when(j + 1 < NSTEP)
        def _():
            _gather_chunk(table_hbm, idx_all, j + 1, bufs[1 - b], sems[1 - b])

        def group_body(g, inner):
            az, azm = inner
            toff = j * CHUNK + g * LANES
            m16 = m_all[pl.ds(toff, LANES)]
            wz16 = wz_all[pl.ds(toff, LANES)]
            for r16 in range(LANES):
                r = g * LANES + r16
                racc = buf[r, pl.ds(0, LANES)]
                for k in range(1, SLICES):
                    racc = racc + buf[r, pl.ds(k * LANES, LANES)]
                az = az + racc * jnp.full((LANES,), wz16[r16], jnp.float32)
                azm = azm + racc * jnp.full((LANES,), m16[r16], jnp.float32)
            return az, azm

        return lax.fori_loop(0, GROUPS, group_body, (acc_z, acc_zm))

    def pair_body(p, carry):
        carry = step(2 * p, 0, carry)
        return step(2 * p + 1, 1, carry)

    zero = jnp.zeros((LANES,), jnp.float32)
    acc_z, acc_zm = lax.fori_loop(0, NSTEP // 2, pair_body, (zero, zero))
    stage_v[0, :] = acc_z
    stage_v[1, :] = acc_zm
    pltpu.sync_copy(stage_v.at[0], pz_hbm.at[wid])
    pltpu.sync_copy(stage_v.at[1], pzm_hbm.at[wid])


@functools.partial(
    pl.kernel,
    out_type=jax.ShapeDtypeStruct((N_PAD, DIM), jnp.float32),
    mesh=_mesh,
    scratch_types=[
        pltpu.VMEM((TOK_PER_W,), jnp.int32),
        pltpu.VMEM((TOK_PER_W,), jnp.float32),
        pltpu.VMEM((CHUNK, DIM), jnp.float32),
        pltpu.VMEM((CHUNK, DIM), jnp.float32),
        pltpu.VMEM((NW, LANES), jnp.float32),
        pltpu.VMEM((NW, LANES), jnp.float32),
        pltpu.SemaphoreType.DMA,
        pltpu.SemaphoreType.DMA,
        pltpu.SemaphoreType.DMA,
        pltpu.SemaphoreType.DMA,
    ],
)
def _scale_kernel(ids_hbm, mult_hbm, table_hbm, pz_hbm, pzm_hbm, out_hbm,
                  idx_all, m_all, buf0, buf1, pz_v, pzm_v,
                  semg0, semg1, semo0, semo1):
    wid = _worker_id()
    base = wid * TOK_PER_W
    pltpu.sync_copy(ids_hbm.at[pl.ds(base, TOK_PER_W)], idx_all)
    pltpu.sync_copy(mult_hbm.at[pl.ds(base, TOK_PER_W)], m_all)

    pltpu.sync_copy(pz_hbm, pz_v)
    pltpu.sync_copy(pzm_hbm, pzm_v)
    vz = pz_v[0, :]
    vzm = pzm_v[0, :]
    for w in range(1, NW):
        vz = vz + pz_v[w, :]
        vzm = vzm + pzm_v[w, :]
    sz = vz[0]
    szm = vzm[0]
    for i in range(1, LANES):
        sz = sz + vz[i]
        szm = szm + vzm[i]
    rsplat = jnp.full((LANES,), sz, jnp.float32) / jnp.full(
        (LANES,), szm, jnp.float32
    )

    bufs = (buf0, buf1)
    semg = (semg0, semg1)
    semo = (semo0, semo1)
    _gather_chunk(table_hbm, idx_all, 0, buf0, semg0)

    def step(j, b, carry):
        buf = bufs[b]
        pltpu.make_async_copy(
            table_hbm.at[idx_all.at[pl.ds(0, CHUNK)]], buf, semg[b]
        ).wait()

        @pl.when(j + 1 < NSTEP)
        def _():
            @pl.when(j >= 1)
            def _():
                pltpu.make_async_copy(
                    bufs[1 - b],
                    out_hbm.at[pl.ds(base, CHUNK)],
                    semo[1 - b],
                ).wait()

            _gather_chunk(table_hbm, idx_all, j + 1, bufs[1 - b], semg[1 - b])

        def group_body(g, inner):
            toff = j * CHUNK + g * LANES
            m16 = m_all[pl.ds(toff, LANES)] * rsplat
            for r16 in range(LANES):
                r = g * LANES + r16
                scale = jnp.full((LANES,), m16[r16], jnp.float32)
                for k in range(SLICES):
                    sl = pl.ds(k * LANES, LANES)
                    buf[r, sl] = buf[r, sl] * scale
            return inner

        lax.fori_loop(0, GROUPS, group_body, 0)
        pltpu.async_copy(buf, out_hbm.at[pl.ds(base + j * CHUNK, CHUNK)], semo[b])
        return carry

    def pair_body(p, carry):
        carry = step(2 * p, 0, carry)
        return step(2 * p + 1, 1, carry)

    lax.fori_loop(0, NSTEP // 2, pair_body, 0)
    pltpu.make_async_copy(
        buf0, out_hbm.at[pl.ds(base, CHUNK)], semo0
    ).wait()
    pltpu.make_async_copy(
        buf1, out_hbm.at[pl.ds(base, CHUNK)], semo1
    ).wait()


def kernel(input_ids, weight, batch_multipliers):
    ids = input_ids.reshape(-1).astype(jnp.int32)
    mult = batch_multipliers.reshape(-1).astype(jnp.float32)
    n_extra = N_PAD - N_TOK
    ids_p = jnp.concatenate([ids, jnp.zeros((n_extra,), jnp.int32)])
    mult_p = jnp.concatenate([mult, jnp.zeros((n_extra,), jnp.float32)])
    valid = jnp.concatenate(
        [jnp.ones((N_TOK,), jnp.float32), jnp.zeros((n_extra,), jnp.float32)]
    )
    pz, pzm = _sums_kernel(ids_p, mult_p, valid, weight)
    out = _scale_kernel(ids_p, mult_p, weight, pz, pzm)
    return out[:N_TOK].reshape(BATCH, SEQ, DIM)


# speedup vs baseline: 1.5369x; 1.4321x over previous
import functools

import jax
import jax.numpy as jnp
from jax import lax
from jax.experimental import pallas as pl
from jax.experimental.pallas import tpu as pltpu
from jax.experimental.pallas import tpu_sc as plsc

VOCAB = 49408
DIM = 1024
BATCH = 256
SEQ = 77
N_TOK = BATCH * SEQ
NC, NS, LANES = 2, 16, 16
NW = NC * NS
N_PAD = 20480
TOK_PER_W = N_PAD // NW
CHUNK = 32
NSTEP = TOK_PER_W // CHUNK
NGROUP = TOK_PER_W // LANES
SLICES = DIM // LANES

_mesh = plsc.VectorSubcoreMesh(core_axis_name="c", subcore_axis_name="s")


def _worker_id():
    return lax.axis_index("s") * NC + lax.axis_index("c")


def _gather_chunk(table_hbm, idx_all, j, buf, sem):
    idx_slice = idx_all.at[pl.ds(j * CHUNK, CHUNK)]
    return pltpu.async_copy(table_hbm.at[idx_slice], buf, sem)


def _row_sum(buf, r):
    accs = [buf[r, pl.ds(k * LANES, LANES)] for k in range(4)]
    for k in range(4, SLICES):
        accs[k % 4] = accs[k % 4] + buf[r, pl.ds(k * LANES, LANES)]
    return (accs[0] + accs[1]) + (accs[2] + accs[3])


@functools.partial(
    pl.kernel,
    out_type=(
        jax.ShapeDtypeStruct((NW, LANES), jnp.float32),
        jax.ShapeDtypeStruct((NW, LANES), jnp.float32),
    ),
    mesh=_mesh,
    scratch_types=[
        pltpu.VMEM((TOK_PER_W,), jnp.int32),
        pltpu.VMEM((TOK_PER_W,), jnp.float32),
        pltpu.VMEM((TOK_PER_W,), jnp.float32),
        pltpu.VMEM((TOK_PER_W * LANES,), jnp.float32),
        pltpu.VMEM((TOK_PER_W * LANES,), jnp.float32),
        pltpu.VMEM((CHUNK, DIM), jnp.float32),
        pltpu.VMEM((CHUNK, DIM), jnp.float32),
        pltpu.VMEM((2, LANES), jnp.float32),
        pltpu.SemaphoreType.DMA,
        pltpu.SemaphoreType.DMA,
    ],
)
def _sums_kernel(ids_hbm, mult_hbm, valid_hbm, table_hbm, pz_hbm, pzm_hbm,
                 idx_all, m_all, wz_all, msp_v, wzsp_v, buf0, buf1,
                 stage_v, sem0, sem1):
    wid = _worker_id()
    base = wid * TOK_PER_W
    pltpu.sync_copy(ids_hbm.at[pl.ds(base, TOK_PER_W)], idx_all)
    pltpu.sync_copy(mult_hbm.at[pl.ds(base, TOK_PER_W)], m_all)
    pltpu.sync_copy(valid_hbm.at[pl.ds(base, TOK_PER_W)], wz_all)

    bufs = (buf0, buf1)
    sems = (sem0, sem1)
    _gather_chunk(table_hbm, idx_all, 0, buf0, sem0)

    def splat_body(g, carry):
        m16 = m_all[pl.ds(g * LANES, LANES)]
        wz16 = wz_all[pl.ds(g * LANES, LANES)]
        for r16 in range(LANES):
            msp_v[pl.ds((g * LANES + r16) * LANES, LANES)] = jnp.full(
                (LANES,), m16[r16], jnp.float32
            )
            wzsp_v[pl.ds((g * LANES + r16) * LANES, LANES)] = jnp.full(
                (LANES,), wz16[r16], jnp.float32
            )
        return carry

    lax.fori_loop(0, NGROUP, splat_body, 0)

    carry = (jnp.zeros((LANES,), jnp.float32), jnp.zeros((LANES,), jnp.float32))
    for j in range(NSTEP):
        b = j % 2
        buf = bufs[b]
        pltpu.make_async_copy(
            table_hbm.at[idx_all.at[pl.ds(0, CHUNK)]], buf, sems[b]
        ).wait()
        if j + 1 < NSTEP:
            _gather_chunk(table_hbm, idx_all, j + 1, bufs[1 - b], sems[1 - b])

        def row_body(r, inner, _j=j, _buf=buf):
            az, azm = inner
            t = _j * CHUNK + r
            racc = _row_sum(_buf, r)
            az = az + racc * wzsp_v[pl.ds(t * LANES, LANES)]
            azm = azm + racc * msp_v[pl.ds(t * LANES, LANES)]
            return az, azm

        carry = lax.fori_loop(0, CHUNK, row_body, carry)

    acc_z, acc_zm = carry
    stage_v[0, :] = acc_z
    stage_v[1, :] = acc_zm
    pltpu.sync_copy(stage_v.at[0], pz_hbm.at[wid])
    pltpu.sync_copy(stage_v.at[1], pzm_hbm.at[wid])


@functools.partial(
    pl.kernel,
    out_type=(
        jax.ShapeDtypeStruct((N_TOK, DIM), jnp.float32),
        jax.ShapeDtypeStruct((CHUNK, DIM), jnp.float32),
    ),
    mesh=_mesh,
    scratch_types=[
        pltpu.VMEM((TOK_PER_W,), jnp.int32),
        pltpu.VMEM((TOK_PER_W,), jnp.float32),
        pltpu.VMEM((TOK_PER_W * LANES,), jnp.float32),
        pltpu.VMEM((CHUNK, DIM), jnp.float32),
        pltpu.VMEM((CHUNK, DIM), jnp.float32),
        pltpu.VMEM((CHUNK, DIM), jnp.float32),
        pltpu.VMEM((NW, LANES), jnp.float32),
        pltpu.VMEM((NW, LANES), jnp.float32),
        pltpu.SemaphoreType.DMA,
        pltpu.SemaphoreType.DMA,
        pltpu.SemaphoreType.DMA,
        pltpu.SemaphoreType.DMA,
        pltpu.SemaphoreType.DMA,
        pltpu.SemaphoreType.DMA,
    ],
)
def _scale_kernel(ids_hbm, mult_hbm, table_hbm, pz_hbm, pzm_hbm,
                  out_hbm, dump_hbm,
                  idx_all, m_all, ssp_v, buf0, buf1, buf2, pz_v, pzm_v,
                  semg0, semg1, semg2, semo0, semo1, semo2):
    wid = _worker_id()
    base = wid * TOK_PER_W
    pltpu.sync_copy(ids_hbm.at[pl.ds(base, TOK_PER_W)], idx_all)
    pltpu.sync_copy(mult_hbm.at[pl.ds(base, TOK_PER_W)], m_all)

    bufs = (buf0, buf1, buf2)
    semg = (semg0, semg1, semg2)
    semo = (semo0, semo1, semo2)
    _gather_chunk(table_hbm, idx_all, 0, bufs[0], semg[0])

    pltpu.sync_copy(pz_hbm, pz_v)
    pltpu.sync_copy(pzm_hbm, pzm_v)
    vz = pz_v[0, :]
    vzm = pzm_v[0, :]
    for w in range(1, NW):
        vz = vz + pz_v[w, :]
        vzm = vzm + pzm_v[w, :]
    sz = vz[0]
    szm = vzm[0]
    for i in range(1, LANES):
        sz = sz + vz[i]
        szm = szm + vzm[i]
    rsplat = jnp.full((LANES,), sz, jnp.float32) / jnp.full(
        (LANES,), szm, jnp.float32
    )

    def splat_body(g, carry):
        m16 = m_all[pl.ds(g * LANES, LANES)] * rsplat
        for r16 in range(LANES):
            ssp_v[pl.ds((g * LANES + r16) * LANES, LANES)] = jnp.full(
                (LANES,), m16[r16], jnp.float32
            )
        return carry

    lax.fori_loop(0, NGROUP, splat_body, 0)

    for j in range(NSTEP):
        b = j % 3
        buf = bufs[b]
        pltpu.make_async_copy(
            table_hbm.at[idx_all.at[pl.ds(0, CHUNK)]], buf, semg[b]
        ).wait()
        if j + 1 < NSTEP:
            bn = (j + 1) % 3
            if j >= 2:
                pltpu.make_async_copy(bufs[bn], dump_hbm, semo[bn]).wait()
            _gather_chunk(table_hbm, idx_all, j + 1, bufs[bn], semg[bn])

        def row_body(r, inner, _j=j, _buf=buf):
            t = _j * CHUNK + r
            scale = ssp_v[pl.ds(t * LANES, LANES)]
            for k in range(SLICES):
                sl = pl.ds(k * LANES, LANES)
                _buf[r, sl] = _buf[r, sl] * scale
            return inner

        lax.fori_loop(0, CHUNK, row_body, 0)

        cbase = base + j * CHUNK

        @pl.when(cbase < N_TOK)
        def _(_buf=buf, _cbase=cbase, _sem=semo[b]):
            pltpu.async_copy(_buf, out_hbm.at[pl.ds(_cbase, CHUNK)], _sem)

        @pl.when(cbase >= N_TOK)
        def _(_buf=buf, _sem=semo[b]):
            pltpu.async_copy(_buf, dump_hbm, _sem)

    pltpu.make_async_copy(bufs[18 % 3], dump_hbm, semo[18 % 3]).wait()
    pltpu.make_async_copy(bufs[19 % 3], dump_hbm, semo[19 % 3]).wait()


def kernel(input_ids, weight, batch_multipliers):
    ids = input_ids.reshape(-1).astype(jnp.int32)
    mult = batch_multipliers.reshape(-1).astype(jnp.float32)
    n_extra = N_PAD - N_TOK
    ids_p = jnp.concatenate([ids, jnp.zeros((n_extra,), jnp.int32)])
    mult_p = jnp.concatenate([mult, jnp.zeros((n_extra,), jnp.float32)])
    valid = jnp.concatenate(
        [jnp.ones((N_TOK,), jnp.float32), jnp.zeros((n_extra,), jnp.float32)]
    )
    pz, pzm = _sums_kernel(ids_p, mult_p, valid, weight)
    out, _ = _scale_kernel(ids_p, mult_p, weight, pz, pzm)
    return out.reshape(BATCH, SEQ, DIM)


# speedup vs baseline: 1.6305x; 1.0609x over previous
import functools

import jax
import jax.numpy as jnp
from jax import lax
from jax.experimental import pallas as pl
from jax.experimental.pallas import tpu as pltpu
from jax.experimental.pallas import tpu_sc as plsc

VOCAB = 49408
DIM = 1024
BATCH = 256
SEQ = 77
N_TOK = BATCH * SEQ
NC, NS, LANES = 2, 16, 16
NW = NC * NS
N_PAD = 20480
TOK_PER_W = N_PAD // NW
CHUNK = 32
NSTEP = TOK_PER_W // CHUNK
NGROUP = TOK_PER_W // LANES
SLICES = DIM // LANES
ROWBLK = 128

_mesh = plsc.VectorSubcoreMesh(core_axis_name="c", subcore_axis_name="s")


def _worker_id():
    return lax.axis_index("s") * NC + lax.axis_index("c")


def _gather_chunk(table_hbm, idx_all, j, buf, sem):
    idx_slice = idx_all.at[pl.ds(j * CHUNK, CHUNK)]
    return pltpu.async_copy(table_hbm.at[idx_slice], buf, sem)


@functools.partial(
    pl.kernel,
    out_type=(
        jax.ShapeDtypeStruct((N_TOK, DIM), jnp.float32),
        jax.ShapeDtypeStruct((NW * LANES,), jnp.float32),
        jax.ShapeDtypeStruct((NW * LANES,), jnp.float32),
    ),
    mesh=_mesh,
    scratch_types=[
        pltpu.VMEM((TOK_PER_W,), jnp.int32),
        pltpu.VMEM((TOK_PER_W,), jnp.float32),
        pltpu.VMEM((TOK_PER_W * LANES,), jnp.float32),
        pltpu.VMEM((CHUNK, DIM), jnp.float32),
        pltpu.VMEM((CHUNK, DIM), jnp.float32),
        pltpu.VMEM((CHUNK, DIM), jnp.float32),
        pltpu.VMEM((2, LANES), jnp.float32),
        pltpu.SemaphoreType.DMA,
        pltpu.SemaphoreType.DMA,
        pltpu.SemaphoreType.DMA,
        pltpu.SemaphoreType.DMA,
        pltpu.SemaphoreType.DMA,
        pltpu.SemaphoreType.DMA,
    ],
)
def _gather_scale_kernel(ids_hbm, mult_hbm, table_hbm,
                         zm_hbm, pz_hbm, pzm_hbm,
                         idx_all, m_all, msp_v, buf0, buf1, buf2,
                         stage_v, semg0, semg1, semg2, semo0, semo1, semo2):
    wid = _worker_id()
    base = wid * TOK_PER_W
    pltpu.sync_copy(ids_hbm.at[pl.ds(base, TOK_PER_W)], idx_all)
    pltpu.sync_copy(mult_hbm.at[pl.ds(base, TOK_PER_W)], m_all)

    bufs = (buf0, buf1, buf2)
    semg = (semg0, semg1, semg2)
    semo = (semo0, semo1, semo2)

    def valid(j):
        return base + j * CHUNK < N_TOK

    @pl.when(valid(0))
    def _():
        _gather_chunk(table_hbm, idx_all, 0, bufs[0], semg[0])

    def splat_body(g, carry):
        m16 = m_all[pl.ds(g * LANES, LANES)]
        for r16 in range(LANES):
            msp_v[pl.ds((g * LANES + r16) * LANES, LANES)] = jnp.full(
                (LANES,), m16[r16], jnp.float32
            )
        return carry

    lax.fori_loop(0, NGROUP, splat_body, 0)

    carry = (jnp.zeros((LANES,), jnp.float32), jnp.zeros((LANES,), jnp.float32))
    for j in range(NSTEP):
        b = j % 3
        buf = bufs[b]

        @pl.when(valid(j))
        def _(_buf=buf, _sem=semg[b]):
            pltpu.make_async_copy(
                table_hbm.at[idx_all.at[pl.ds(0, CHUNK)]], _buf, _sem
            ).wait()

        if j + 1 < NSTEP:
            bn = (j + 1) % 3

            @pl.when(valid(j + 1))
            def _(_j=j, _bn=bn):
                if _j >= 2:
                    pltpu.make_async_copy(
                        bufs[_bn], zm_hbm.at[pl.ds(0, CHUNK)], semo[_bn]
                    ).wait()
                _gather_chunk(table_hbm, idx_all, _j + 1, bufs[_bn], semg[_bn])

        vgate = jnp.where(valid(j), 1.0, 0.0).astype(jnp.float32)
        vsplat = jnp.full((LANES,), vgate, jnp.float32)

        def row_body(r, inner, _j=j, _buf=buf, _vsplat=vsplat):
            az, azm = inner
            t = _j * CHUNK + r
            msplat = msp_v[pl.ds(t * LANES, LANES)]
            accs = [None, None, None, None]
            for k in range(SLICES):
                sl = pl.ds(k * LANES, LANES)
                v = _buf[r, sl]
                if k < 4:
                    accs[k] = v
                else:
                    accs[k % 4] = accs[k % 4] + v
                _buf[r, sl] = v * msplat
            racc = (accs[0] + accs[1]) + (accs[2] + accs[3])
            az = az + racc * _vsplat
            azm = azm + racc * msplat
            return az, azm

        carry = lax.fori_loop(0, CHUNK, row_body, carry)

        @pl.when(valid(j))
        def _(_buf=buf, _j=j, _sem=semo[b]):
            pltpu.async_copy(
                _buf, zm_hbm.at[pl.ds(base + _j * CHUNK, CHUNK)], _sem
            )

    acc_z, acc_zm = carry
    stage_v[0, :] = acc_z
    stage_v[1, :] = acc_zm
    pltpu.sync_copy(stage_v.at[0], pz_hbm.at[pl.ds(wid * LANES, LANES)])
    pltpu.sync_copy(stage_v.at[1], pzm_hbm.at[pl.ds(wid * LANES, LANES)])

    @pl.when(valid(0))
    def _():
        for b in range(3):
            pltpu.make_async_copy(
                bufs[b], zm_hbm.at[pl.ds(0, CHUNK)], semo[b]
            ).wait()


def _ratio_scale_body(x_ref, pz_ref, pzm_ref, o_ref):
    sz = jnp.sum(pz_ref[...])
    szm = jnp.sum(pzm_ref[...])
    o_ref[...] = x_ref[...] * (sz / szm)


_ratio_scale = pl.pallas_call(
    _ratio_scale_body,
    grid=(N_TOK // ROWBLK,),
    in_specs=[
        pl.BlockSpec((ROWBLK, DIM), lambda i: (i, 0)),
        pl.BlockSpec((4, 128), lambda i: (0, 0)),
        pl.BlockSpec((4, 128), lambda i: (0, 0)),
    ],
    out_specs=pl.BlockSpec((ROWBLK, DIM), lambda i: (i, 0)),
    out_shape=jax.ShapeDtypeStruct((N_TOK, DIM), jnp.float32),
    input_output_aliases={0: 0},
)


def kernel(input_ids, weight, batch_multipliers):
    ids = input_ids.reshape(-1).astype(jnp.int32)
    mult = batch_multipliers.reshape(-1).astype(jnp.float32)
    n_extra = N_PAD - N_TOK
    ids_p = jnp.concatenate([ids, jnp.zeros((n_extra,), jnp.int32)])
    mult_p = jnp.concatenate([mult, jnp.zeros((n_extra,), jnp.float32)])
    zm, pz, pzm = _gather_scale_kernel(ids_p, mult_p, weight)
    out = _ratio_scale(zm, pz.reshape(4, 128), pzm.reshape(4, 128))
    return out.reshape(BATCH, SEQ, DIM)


# speedup vs baseline: 1.8314x; 1.1233x over previous
import functools

import jax
import jax.numpy as jnp
from jax import lax
from jax.experimental import pallas as pl
from jax.experimental.pallas import tpu as pltpu
from jax.experimental.pallas import tpu_sc as plsc

VOCAB = 49408
DIM = 1024
BATCH = 256
SEQ = 77
N_TOK = BATCH * SEQ
NC, NS, LANES = 2, 16, 16
NW = NC * NS
N_PAD = 20480
TOK_PER_W = N_PAD // NW
CHUNK = 32
NSTEP = TOK_PER_W // CHUNK
NGROUP = TOK_PER_W // LANES
SLICES = DIM // LANES
ROWBLK = 128

_mesh = plsc.VectorSubcoreMesh(core_axis_name="c", subcore_axis_name="s")


def _worker_id():
    return lax.axis_index("s") * NC + lax.axis_index("c")


def _gather_chunk(table_hbm, idx_all, j, buf, sem):
    idx_slice = idx_all.at[pl.ds(j * CHUNK, CHUNK)]
    return pltpu.async_copy(table_hbm.at[idx_slice], buf, sem)


@functools.partial(
    pl.kernel,
    out_type=(
        jax.ShapeDtypeStruct((N_TOK, DIM), jnp.float32),
        jax.ShapeDtypeStruct((NW * LANES,), jnp.float32),
        jax.ShapeDtypeStruct((NW * LANES,), jnp.float32),
    ),
    mesh=_mesh,
    scratch_types=[
        pltpu.VMEM((TOK_PER_W,), jnp.int32),
        pltpu.VMEM((TOK_PER_W,), jnp.float32),
        pltpu.VMEM((TOK_PER_W * LANES,), jnp.float32),
        pltpu.VMEM((CHUNK, DIM), jnp.float32),
        pltpu.VMEM((CHUNK, DIM), jnp.float32),
        pltpu.VMEM((CHUNK, DIM), jnp.float32),
        pltpu.VMEM((2, LANES), jnp.float32),
        pltpu.SemaphoreType.DMA,
        pltpu.SemaphoreType.DMA,
        pltpu.SemaphoreType.DMA,
        pltpu.SemaphoreType.DMA,
        pltpu.SemaphoreType.DMA,
        pltpu.SemaphoreType.DMA,
    ],
)
def _gather_scale_kernel(ids_hbm, mult_hbm, table_hbm,
                         zm_hbm, pz_hbm, pzm_hbm,
                         idx_all, m_all, msp_v, buf0, buf1, buf2,
                         stage_v, semg0, semg1, semg2, semo0, semo1, semo2):
    wid = _worker_id()
    base = wid * TOK_PER_W
    pltpu.sync_copy(ids_hbm.at[pl.ds(base, TOK_PER_W)], idx_all)
    pltpu.sync_copy(mult_hbm.at[pl.ds(base, TOK_PER_W)], m_all)

    bufs = (buf0, buf1, buf2)
    semg = (semg0, semg1, semg2)
    semo = (semo0, semo1, semo2)

    def valid(j):
        return base + j * CHUNK < N_TOK

    @pl.when(valid(0))
    def _():
        _gather_chunk(table_hbm, idx_all, 0, bufs[0], semg[0])

    def splat_body(g, carry):
        m16 = m_all[pl.ds(g * LANES, LANES)]
        for r16 in range(LANES):
            msp_v[pl.ds((g * LANES + r16) * LANES, LANES)] = jnp.full(
                (LANES,), m16[r16], jnp.float32
            )
        return carry

    lax.fori_loop(0, NGROUP, splat_body, 0)

    carry = (jnp.zeros((LANES,), jnp.float32), jnp.zeros((LANES,), jnp.float32))
    for j in range(NSTEP):
        b = j % 3
        buf = bufs[b]

        @pl.when(valid(j))
        def _(_buf=buf, _sem=semg[b]):
            pltpu.make_async_copy(
                table_hbm.at[idx_all.at[pl.ds(0, CHUNK)]], _buf, _sem
            ).wait()

        if j + 1 < NSTEP:
            bn = (j + 1) % 3

            @pl.when(valid(j + 1))
            def _(_j=j, _bn=bn):
                if _j >= 2:
                    pltpu.make_async_copy(
                        bufs[_bn], zm_hbm.at[pl.ds(0, CHUNK)], semo[_bn]
                    ).wait()
                _gather_chunk(table_hbm, idx_all, _j + 1, bufs[_bn], semg[_bn])

        vgate = jnp.where(valid(j), 1.0, 0.0).astype(jnp.float32)
        vsplat = jnp.full((LANES,), vgate, jnp.float32)

        def row_body(r, inner, _j=j, _buf=buf, _vsplat=vsplat):
            az, azm = inner
            t = _j * CHUNK + r
            msplat = msp_v[pl.ds(t * LANES, LANES)]
            accs = [None, None, None, None]
            for k in range(SLICES):
                sl = pl.ds(k * LANES, LANES)
                v = _buf[r, sl]
                if k < 4:
                    accs[k] = v
                else:
                    accs[k % 4] = accs[k % 4] + v
                _buf[r, sl] = v * msplat
            racc = (accs[0] + accs[1]) + (accs[2] + accs[3])
            az = az + racc * _vsplat
            azm = azm + racc * msplat
            return az, azm

        carry = lax.fori_loop(0, CHUNK, row_body, carry)

        @pl.when(valid(j))
        def _(_buf=buf, _j=j, _sem=semo[b]):
            pltpu.async_copy(
                _buf, zm_hbm.at[pl.ds(base + _j * CHUNK, CHUNK)], _sem
            )

    acc_z, acc_zm = carry
    stage_v[0, :] = acc_z
    stage_v[1, :] = acc_zm
    pltpu.sync_copy(stage_v.at[0], pz_hbm.at[pl.ds(wid * LANES, LANES)])
    pltpu.sync_copy(stage_v.at[1], pzm_hbm.at[pl.ds(wid * LANES, LANES)])

    @pl.when(valid(0))
    def _():
        for b in range(3):
            pltpu.make_async_copy(
                bufs[b], zm_hbm.at[pl.ds(0, CHUNK)], semo[b]
            ).wait()


def _ratio_scale_body(x_ref, pz_ref, pzm_ref, o_ref):
    sz = jnp.sum(pz_ref[...])
    szm = jnp.sum(pzm_ref[...])
    o_ref[...] = x_ref[...] * (sz / szm)


BBLK = 4


_ratio_scale = pl.pallas_call(
    _ratio_scale_body,
    grid=(BATCH // BBLK,),
    in_specs=[
        pl.BlockSpec((BBLK, SEQ, DIM), lambda i: (i, 0, 0)),
        pl.BlockSpec((4, 128), lambda i: (0, 0)),
        pl.BlockSpec((4, 128), lambda i: (0, 0)),
    ],
    out_specs=pl.BlockSpec((BBLK, SEQ, DIM), lambda i: (i, 0, 0)),
    out_shape=jax.ShapeDtypeStruct((BATCH, SEQ, DIM), jnp.float32),
    input_output_aliases={0: 0},
)


def kernel(input_ids, weight, batch_multipliers):
    ids = input_ids.reshape(-1).astype(jnp.int32)
    mult = batch_multipliers.reshape(-1).astype(jnp.float32)
    n_extra = N_PAD - N_TOK
    ids_p = jnp.concatenate([ids, jnp.zeros((n_extra,), jnp.int32)])
    mult_p = jnp.concatenate([mult, jnp.zeros((n_extra,), jnp.float32)])
    zm, pz, pzm = _gather_scale_kernel(ids_p, mult_p, weight)
    zm3 = zm.reshape(BATCH, SEQ, DIM)
    return _ratio_scale(zm3, pz.reshape(4, 128), pzm.reshape(4, 128))


# speedup vs baseline: 1.9103x; 1.0431x over previous
import functools

import jax
import jax.numpy as jnp
from jax import lax
from jax.experimental import pallas as pl
from jax.experimental.pallas import tpu as pltpu
from jax.experimental.pallas import tpu_sc as plsc

VOCAB = 49408
DIM = 1024
BATCH = 256
SEQ = 77
N_TOK = BATCH * SEQ
NC, NS, LANES = 2, 16, 16
NW = NC * NS
N_PAD = 20480
TOK_PER_W = N_PAD // NW
CHUNK = 32
NSTEP = TOK_PER_W // CHUNK
NGROUP = TOK_PER_W // LANES
SLICES = DIM // LANES
ROWBLK = 128

_mesh = plsc.VectorSubcoreMesh(core_axis_name="c", subcore_axis_name="s")


def _worker_id():
    return lax.axis_index("s") * NC + lax.axis_index("c")


def _gather_chunk(table_hbm, idx_all, j, buf, sem):
    idx_slice = idx_all.at[pl.ds(j * CHUNK, CHUNK)]
    return pltpu.async_copy(table_hbm.at[idx_slice], buf, sem)


@functools.partial(
    pl.kernel,
    out_type=(
        jax.ShapeDtypeStruct((N_TOK, DIM), jnp.float32),
        jax.ShapeDtypeStruct((NW * LANES,), jnp.float32),
        jax.ShapeDtypeStruct((NW * LANES,), jnp.float32),
    ),
    mesh=_mesh,
    scratch_types=[
        pltpu.VMEM((TOK_PER_W,), jnp.int32),
        pltpu.VMEM((TOK_PER_W,), jnp.float32),
        pltpu.VMEM((TOK_PER_W * LANES,), jnp.float32),
        pltpu.VMEM((CHUNK, DIM), jnp.float32),
        pltpu.VMEM((CHUNK, DIM), jnp.float32),
        pltpu.VMEM((CHUNK, DIM), jnp.float32),
        pltpu.VMEM((2, LANES), jnp.float32),
        pltpu.SemaphoreType.DMA,
        pltpu.SemaphoreType.DMA,
        pltpu.SemaphoreType.DMA,
        pltpu.SemaphoreType.DMA,
        pltpu.SemaphoreType.DMA,
        pltpu.SemaphoreType.DMA,
    ],
)
def _gather_scale_kernel(ids_hbm, mult_hbm, table_hbm,
                         zm_hbm, pz_hbm, pzm_hbm,
                         idx_all, m_all, msp_v, buf0, buf1, buf2,
                         stage_v, semg0, semg1, semg2, semo0, semo1, semo2):
    wid = _worker_id()
    base = wid * TOK_PER_W
    pltpu.sync_copy(ids_hbm.at[pl.ds(base, TOK_PER_W)], idx_all)
    pltpu.sync_copy(mult_hbm.at[pl.ds(base, TOK_PER_W)], m_all)

    bufs = (buf0, buf1, buf2)
    semg = (semg0, semg1, semg2)
    semo = (semo0, semo1, semo2)

    def valid(j):
        return base + j * CHUNK < N_TOK

    @pl.when(valid(0))
    def _():
        _gather_chunk(table_hbm, idx_all, 0, bufs[0], semg[0])

    def splat_body(g, carry):
        m16 = m_all[pl.ds(g * LANES, LANES)]
        for r16 in range(LANES):
            msp_v[pl.ds((g * LANES + r16) * LANES, LANES)] = jnp.full(
                (LANES,), m16[r16], jnp.float32
            )
        return carry

    lax.fori_loop(0, NGROUP, splat_body, 0)

    carry = (jnp.zeros((LANES,), jnp.float32), jnp.zeros((LANES,), jnp.float32))
    for j in range(NSTEP):
        b = j % 3
        buf = bufs[b]

        @pl.when(valid(j))
        def _(_buf=buf, _sem=semg[b]):
            pltpu.make_async_copy(
                table_hbm.at[idx_all.at[pl.ds(0, CHUNK)]], _buf, _sem
            ).wait()

        if j + 1 < NSTEP:
            bn = (j + 1) % 3

            @pl.when(valid(j + 1))
            def _(_j=j, _bn=bn):
                if _j >= 2:
                    pltpu.make_async_copy(
                        bufs[_bn], zm_hbm.at[pl.ds(0, CHUNK)], semo[_bn]
                    ).wait()
                _gather_chunk(table_hbm, idx_all, _j + 1, bufs[_bn], semg[_bn])

        vgate = jnp.where(valid(j), 1.0, 0.0).astype(jnp.float32)
        vsplat = jnp.full((LANES,), vgate, jnp.float32)

        def row_body(r, inner, _j=j, _buf=buf, _vsplat=vsplat):
            az, azm = inner
            t = _j * CHUNK + r
            msplat = msp_v[pl.ds(t * LANES, LANES)]
            accs = [None, None, None, None]
            for k in range(SLICES):
                sl = pl.ds(k * LANES, LANES)
                v = _buf[r, sl]
                if k < 4:
                    accs[k] = v
                else:
                    accs[k % 4] = accs[k % 4] + v
                _buf[r, sl] = v * msplat
            racc = (accs[0] + accs[1]) + (accs[2] + accs[3])
            az = az + racc * _vsplat
            azm = azm + racc * msplat
            return az, azm

        carry = lax.fori_loop(0, CHUNK, row_body, carry)

        @pl.when(valid(j))
        def _(_buf=buf, _j=j, _sem=semo[b]):
            pltpu.async_copy(
                _buf, zm_hbm.at[pl.ds(base + _j * CHUNK, CHUNK)], _sem
            )

    acc_z, acc_zm = carry
    stage_v[0, :] = acc_z
    stage_v[1, :] = acc_zm
    pltpu.sync_copy(stage_v.at[0], pz_hbm.at[pl.ds(wid * LANES, LANES)])
    pltpu.sync_copy(stage_v.at[1], pzm_hbm.at[pl.ds(wid * LANES, LANES)])

    @pl.when(valid(0))
    def _():
        for b in range(3):
            pltpu.make_async_copy(
                bufs[b], zm_hbm.at[pl.ds(0, CHUNK)], semo[b]
            ).wait()


R_PER_W = N_TOK // NW
RCHUNK = 56
RNSTEP = R_PER_W // RCHUNK


@functools.partial(
    pl.kernel,
    out_type=jax.ShapeDtypeStruct((N_TOK, DIM), jnp.float32),
    mesh=_mesh,
    scratch_types=[
        pltpu.VMEM((NW * LANES,), jnp.float32),
        pltpu.VMEM((NW * LANES,), jnp.float32),
        pltpu.VMEM((RCHUNK, DIM), jnp.float32),
        pltpu.VMEM((RCHUNK, DIM), jnp.float32),
        pltpu.SemaphoreType.DMA,
        pltpu.SemaphoreType.DMA,
        pltpu.SemaphoreType.DMA,
        pltpu.SemaphoreType.DMA,
    ],
)
def _ratio_scale_kernel(zm_hbm, pz_hbm, pzm_hbm, out_hbm,
                        pz_v, pzm_v, buf0, buf1,
                        semg0, semg1, semo0, semo1):
    wid = _worker_id()
    base = wid * R_PER_W
    bufs = (buf0, buf1)
    semg = (semg0, semg1)
    semo = (semo0, semo1)

    pltpu.async_copy(zm_hbm.at[pl.ds(base, RCHUNK)], bufs[0], semg[0])

    pltpu.sync_copy(pz_hbm, pz_v)
    pltpu.sync_copy(pzm_hbm, pzm_v)
    vz = pz_v[pl.ds(0, LANES)]
    vzm = pzm_v[pl.ds(0, LANES)]
    for w in range(1, NW):
        vz = vz + pz_v[pl.ds(w * LANES, LANES)]
        vzm = vzm + pzm_v[pl.ds(w * LANES, LANES)]
    sz = vz[0]
    szm = vzm[0]
    for i in range(1, LANES):
        sz = sz + vz[i]
        szm = szm + vzm[i]
    rsplat = jnp.full((LANES,), sz, jnp.float32) / jnp.full(
        (LANES,), szm, jnp.float32
    )

    for j in range(RNSTEP):
        b = j % 2
        buf = bufs[b]
        pltpu.make_async_copy(
            zm_hbm.at[pl.ds(0, RCHUNK)], buf, semg[b]
        ).wait()
        if j + 1 < RNSTEP:
            bn = 1 - b
            if j >= 1:
                pltpu.make_async_copy(
                    bufs[bn], out_hbm.at[pl.ds(0, RCHUNK)], semo[bn]
                ).wait()
            pltpu.async_copy(
                zm_hbm.at[pl.ds(base + (j + 1) * RCHUNK, RCHUNK)],
                bufs[bn], semg[bn],
            )

        def row_body(r, inner, _buf=buf):
            for k in range(SLICES):
                sl = pl.ds(k * LANES, LANES)
                _buf[r, sl] = _buf[r, sl] * rsplat
            return inner

        lax.fori_loop(0, RCHUNK, row_body, 0)
        pltpu.async_copy(
            buf, out_hbm.at[pl.ds(base + j * RCHUNK, RCHUNK)], semo[b]
        )

    for b in range(2):
        pltpu.make_async_copy(
            bufs[b], out_hbm.at[pl.ds(0, RCHUNK)], semo[b]
        ).wait()


def kernel(input_ids, weight, batch_multipliers):
    ids = input_ids.reshape(-1).astype(jnp.int32)
    mult = batch_multipliers.reshape(-1).astype(jnp.float32)
    n_extra = N_PAD - N_TOK
    ids_p = jnp.concatenate([ids, jnp.zeros((n_extra,), jnp.int32)])
    mult_p = jnp.concatenate([mult, jnp.zeros((n_extra,), jnp.float32)])
    zm, pz, pzm = _gather_scale_kernel(ids_p, mult_p, weight)
    out = _ratio_scale_kernel(zm, pz, pzm)
    return out.reshape(BATCH, SEQ, DIM)
